# baseline (speedup 1.0000x reference)
import functools

import jax
import jax.numpy as jnp
from jax import lax
from jax.experimental import pallas as pl
from jax.experimental.pallas import tpu as pltpu
from jax.experimental.pallas import tpu_sc as plsc

_NC = 2
_NS = 16
_NW = _NC * _NS
_BLK = 8


@functools.lru_cache(maxsize=None)
def _make_emb(b, l, v, d):
    per_w = b // _NW
    nblk = per_w // _BLK
    lp1 = l + 1
    lpad = (lp1 + 7) // 8 * 8
    mesh = plsc.VectorSubcoreMesh(core_axis_name="c", subcore_axis_name="s")

    @functools.partial(
        pl.kernel,
        mesh=mesh,
        out_type=jax.ShapeDtypeStruct((b, lpad, d), jnp.float32),
        scratch_types=[
            pltpu.VMEM((2, _BLK, l), jnp.int32),
            pltpu.VMEM((2, _BLK, lpad, d), jnp.float32),
            pltpu.VMEM((1, d), jnp.float32),
            pltpu.SemaphoreType.DMA,
            pltpu.SemaphoreType.DMA,
            pltpu.SemaphoreType.DMA,
            pltpu.SemaphoreType.DMA,
            pltpu.SemaphoreType.DMA,
        ],
        compiler_params=pltpu.CompilerParams(use_tc_tiling_on_sc=False),
    )
    def emb(in_hbm, table_hbm, cls_hbm, out_hbm, idx_v, rows_v, crow_v,
            sem_c, g0, g1, o0, o1):
        wid = lax.axis_index("s") * _NC + lax.axis_index("c")
        base = wid * per_w
        sem_g = [g0, g1]
        sem_o = [o0, o1]

        pltpu.async_copy(cls_hbm, crow_v, sem_c).wait()
        c0 = crow_v[0, pl.ds(0, 16)]
        c1 = crow_v[0, pl.ds(16, 16)]

        def gath(j, s):
            return pltpu.make_async_copy(
                table_hbm.at[idx_v.at[s, j]],
                rows_v.at[s, j, pl.ds(1, l)],
                sem_g[s])

        def out_copy(t, s):
            return pltpu.make_async_copy(
                rows_v.at[s],
                out_hbm.at[pl.ds(base + t * _BLK, _BLK)],
                sem_o[s])

        def body(tt, carry):
            for s in range(2):
                t = 2 * tt + s
                @pl.when(tt >= 1)
                def _():
                    out_copy(t - 2, s).wait()
                pltpu.sync_copy(in_hbm.at[pl.ds(base + t * _BLK, _BLK)],
                                idx_v.at[s])
                for j in range(_BLK):
                    rows_v[s, j, 0, pl.ds(0, 16)] = c0
                    rows_v[s, j, 0, pl.ds(16, 16)] = c1
                for j in range(_BLK):
                    gath(j, s).start()
                if s == 0:
                    @pl.when(tt >= 1)
                    def _():
                        for j in range(_BLK):
                            gath(j, 1).wait()
                        out_copy(t - 1, 1).start()
                else:
                    for j in range(_BLK):
                        gath(j, 0).wait()
                    out_copy(t - 1, 0).start()
            return carry

        lax.fori_loop(0, nblk // 2, body, 0)

        for j in range(_BLK):
            gath(j, 1).wait()
        out_copy(nblk - 1, 1).start()
        out_copy(nblk - 2, 0).wait()
        out_copy(nblk - 1, 1).wait()

    return emb


def kernel(inputs, table):
    b, l = inputs.shape
    v, d = table.shape
    out = _make_emb(b, l, v, d)(
        inputs.astype(jnp.int32), table[:v - 1], table[v - 1:])
    return out[:, :l + 1, :]
